# baseline (speedup 1.0000x reference)
import functools

import jax
import jax.numpy as jnp
from jax import lax
from jax.experimental import pallas as pl
from jax.experimental.pallas import tpu as pltpu
from jax.experimental.pallas import tpu_sc as plsc

N = 10000
E = 320000
D = 128
H = 4
HP = 16

NC = 2
NS = 16
NW = NC * NS
EB = 128
NBW = 80
NBH = NBW // 4
EPAD = NW * NBW * EB
NP = 10240
RPS = NP // NS

_R1 = 1024
_G1 = NP // _R1

_mesh = plsc.VectorSubcoreMesh(core_axis_name="c", subcore_axis_name="s")
_sc_params = pltpu.CompilerParams(use_tc_tiling_on_sc=False,
                                  needs_layout_passes=False)


def _k1_body(x_ref, wt_ref, attl_ref, attr_ref, g_ref, b_ref,
             xl_ref, al_ref, ar_ref, cal_ref, car_ref):
    i = pl.program_id(0)
    x = x_ref[...]
    mu = jnp.mean(x, axis=-1, keepdims=True)
    xc = x - mu
    var = jnp.mean(xc * xc, axis=-1, keepdims=True)
    h = g_ref[...] * xc * lax.rsqrt(var + 1e-5) + b_ref[...]
    xl = jnp.dot(h, wt_ref[...], preferred_element_type=jnp.float32)
    xl_ref[...] = xl
    al = jnp.dot(xl, attl_ref[...], preferred_element_type=jnp.float32)
    ar = jnp.dot(xl, attr_ref[...], preferred_element_type=jnp.float32)
    al_ref[...] = al
    ar_ref[...] = ar
    bl = jnp.max(al, axis=0, keepdims=True)
    br = jnp.max(ar, axis=0, keepdims=True)

    @pl.when(i == 0)
    def _():
        cal_ref[...] = bl
        car_ref[...] = br

    @pl.when(i > 0)
    def _():
        cal_ref[...] = jnp.maximum(cal_ref[...], bl)
        car_ref[...] = jnp.maximum(car_ref[...], br)


_k1 = pl.pallas_call(
    _k1_body,
    grid=(_G1,),
    in_specs=[
        pl.BlockSpec((_R1, D), lambda i: (i, 0)),
        pl.BlockSpec((D, D), lambda i: (0, 0)),
        pl.BlockSpec((D, HP), lambda i: (0, 0)),
        pl.BlockSpec((D, HP), lambda i: (0, 0)),
        pl.BlockSpec((1, D), lambda i: (0, 0)),
        pl.BlockSpec((1, D), lambda i: (0, 0)),
    ],
    out_specs=[
        pl.BlockSpec((_R1, D), lambda i: (i, 0)),
        pl.BlockSpec((_R1, HP), lambda i: (i, 0)),
        pl.BlockSpec((_R1, HP), lambda i: (i, 0)),
        pl.BlockSpec((1, HP), lambda i: (0, 0)),
        pl.BlockSpec((1, HP), lambda i: (0, 0)),
    ],
    out_shape=[
        jax.ShapeDtypeStruct((NP, D), jnp.float32),
        jax.ShapeDtypeStruct((NP, HP), jnp.float32),
        jax.ShapeDtypeStruct((NP, HP), jnp.float32),
        jax.ShapeDtypeStruct((1, HP), jnp.float32),
        jax.ShapeDtypeStruct((1, HP), jnp.float32),
    ],
)


@functools.partial(
    pl.kernel,
    out_type=[
        jax.ShapeDtypeStruct((EPAD, HP), jnp.float32),
        jax.ShapeDtypeStruct((NC, NP, HP), jnp.float32),
    ],
    mesh=_mesh,
    scratch_types=[
        pltpu.VMEM((NBW, EB), jnp.int32),
        pltpu.VMEM((NBW, EB), jnp.int32),
        pltpu.VMEM((EB, HP), jnp.float32), pltpu.VMEM((EB, HP), jnp.float32),
        pltpu.VMEM((EB, HP), jnp.float32), pltpu.VMEM((EB, HP), jnp.float32),
        pltpu.VMEM((16,), jnp.float32),
        pltpu.VMEM((16,), jnp.float32),
        pltpu.VMEM_SHARED((NP, HP), jnp.float32),
        pltpu.SemaphoreType.DMA, pltpu.SemaphoreType.DMA,
    ],
    compiler_params=_sc_params,
)
def _k2(dst_hbm, src_hbm, al_hbm, ar_hbm, cal_hbm, car_hbm, z16_hbm,
        ex_hbm, dpart_hbm,
        dstv, srcv, gal0, gal1, gar0, gar1,
        calv, carv, dshared, sem0, sem1):
    cid = lax.axis_index("c")
    sid = lax.axis_index("s")
    wid = sid * NC + cid
    gal, gar = (gal0, gal1), (gar0, gar1)
    sem = (sem0, sem1)
    pltpu.sync_copy(z16_hbm.at[pl.ds(sid * RPS, RPS)],
                    dshared.at[pl.ds(sid * RPS, RPS)])
    pltpu.sync_copy(cal_hbm.at[0], calv)
    pltpu.sync_copy(car_hbm.at[0], carv)
    pltpu.sync_copy(dst_hbm.at[pl.ds(wid * NBW, NBW)], dstv)
    pltpu.sync_copy(src_hbm.at[pl.ds(wid * NBW, NBW)], srcv)
    plsc.subcore_barrier()
    c = jnp.maximum(calv[...] + carv[...], 0.0)
    e_base = wid * NBW * EB

    def issue(g, b):
        pltpu.async_copy(al_hbm.at[dstv.at[g]], gal[b], sem[b])
        pltpu.async_copy(ar_hbm.at[srcv.at[g]], gar[b], sem[b])

    def wait_in(g, b):
        pltpu.make_async_copy(al_hbm.at[dstv.at[g]], gal[b], sem[b]).wait()
        pltpu.make_async_copy(ar_hbm.at[srcv.at[g]], gar[b], sem[b]).wait()

    issue(0, 0)

    @pl.loop(0, NBW, step=2)
    def _(g):
        for b in (0, 1):
            gg = g + b
            wait_in(gg, b)

            @pl.when(gg + 1 < NBW)
            def _():
                issue(gg + 1, 1 - b)

            galb, garb = gal[b], gar[b]

            @plsc.parallel_loop(0, EB, unroll=4)
            def _(j):
                s = galb[j, :] + garb[j, :]
                s = jnp.maximum(s, 0.2 * s)
                galb[j, :] = jnp.exp(s - c)

            pltpu.sync_copy(galb, ex_hbm.at[pl.ds(e_base + gg * EB, EB)])
            pltpu.sync_copy(galb, dshared.at[dstv.at[gg]], add=True)

    plsc.subcore_barrier()
    pltpu.sync_copy(dshared.at[pl.ds(sid * RPS, RPS)],
                    dpart_hbm.at[cid, pl.ds(sid * RPS, RPS)])


def _k3_body(dp_ref, rcp_ref):
    d = dp_ref[0] + dp_ref[1]
    lanes = lax.broadcasted_iota(jnp.int32, (1, HP), 1)
    rcp_ref[...] = jnp.where(lanes < H, 1.0 / (d + 1e-16), 0.0)


_k3 = pl.pallas_call(
    _k3_body,
    out_shape=jax.ShapeDtypeStruct((NP, HP), jnp.float32),
)


@functools.partial(
    pl.kernel,
    out_type=jax.ShapeDtypeStruct((NC, NP, D), jnp.float32),
    mesh=_mesh,
    scratch_types=[
        pltpu.VMEM((EB,), jnp.int32), pltpu.VMEM((EB,), jnp.int32),
        pltpu.VMEM((EB,), jnp.int32), pltpu.VMEM((EB,), jnp.int32),
        pltpu.VMEM((EB, HP), jnp.float32), pltpu.VMEM((EB, HP), jnp.float32),
        pltpu.VMEM((EB, HP), jnp.float32), pltpu.VMEM((EB, HP), jnp.float32),
        pltpu.VMEM((EB, D), jnp.float32), pltpu.VMEM((EB, D), jnp.float32),
        pltpu.VMEM((EB,), jnp.float32),
        pltpu.VMEM_SHARED((NP, D), jnp.float32),
        pltpu.SemaphoreType.DMA, pltpu.SemaphoreType.DMA,
    ],
    compiler_params=_sc_params,
)
def _k4(dst_hbm, src_hbm, xl_hbm, ex_hbm, rcp_hbm, znd_hbm,
        opart_hbm,
        dsti0, dsti1, srci0, srci1, exb0, exb1, rcpb0, rcpb1,
        rows0, rows1, wbuf, oshared, sem0, sem1):
    cid = lax.axis_index("c")
    sid = lax.axis_index("s")
    wid = sid * NC + cid
    dsti, srci = (dsti0, dsti1), (srci0, srci1)
    exb, rcpb, rows = (exb0, exb1), (rcpb0, rcpb1), (rows0, rows1)
    sem = (sem0, sem1)
    pltpu.sync_copy(znd_hbm.at[pl.ds(sid * RPS, RPS)],
                    oshared.at[pl.ds(sid * RPS, RPS)])
    plsc.subcore_barrier()
    lane = lax.iota(jnp.int32, 16)
    colh = [jnp.full((16,), h, jnp.int32) for h in range(H)]
    blk0 = wid * NBW
    e_base = blk0 * EB

    def issue(g, b):
        pltpu.sync_copy(dst_hbm.at[blk0 + g], dsti[b])
        pltpu.sync_copy(src_hbm.at[blk0 + g], srci[b])
        pltpu.async_copy(ex_hbm.at[pl.ds(e_base + g * EB, EB)], exb[b], sem[b])
        pltpu.async_copy(rcp_hbm.at[dsti[b]], rcpb[b], sem[b])
        pltpu.async_copy(xl_hbm.at[srci[b]], rows[b], sem[b])

    def wait_in(g, b):
        pltpu.make_async_copy(ex_hbm.at[pl.ds(e_base + g * EB, EB)], exb[b],
                              sem[b]).wait()
        pltpu.make_async_copy(rcp_hbm.at[dsti[b]], rcpb[b], sem[b]).wait()
        pltpu.make_async_copy(xl_hbm.at[srci[b]], rows[b], sem[b]).wait()

    issue(0, 0)

    @pl.loop(0, NBW, step=2)
    def _(g):
        for b in (0, 1):
            gg = g + b
            wait_in(gg, b)

            @pl.when(gg + 1 < NBW)
            def _():
                issue(gg + 1, 1 - b)

            exbb, rcpbb, rowsb = exb[b], rcpb[b], rows[b]

            @pl.loop(0, EB, step=16)
            def _(j):
                rowi = lane + j
                acc = (plsc.load_gather(exbb, [rowi, colh[0]])
                       * plsc.load_gather(rcpbb, [rowi, colh[0]]))
                for h in range(1, H):
                    acc = acc + (plsc.load_gather(exbb, [rowi, colh[h]])
                                 * plsc.load_gather(rcpbb, [rowi, colh[h]]))
                wbuf[pl.ds(j, 16)] = 0.25 * acc

            @pl.loop(0, EB, unroll=2)
            def _(j):
                wv = plsc.load_gather(
                    wbuf, [jnp.full((16,), 0, jnp.int32) + j])
                for k in range(D // 16):
                    rowsb[j, pl.ds(k * 16, 16)] = (
                        rowsb[j, pl.ds(k * 16, 16)] * wv)

            pltpu.sync_copy(rowsb, oshared.at[dsti[b]], add=True)

    plsc.subcore_barrier()
    pltpu.sync_copy(oshared.at[pl.ds(sid * RPS, RPS)],
                    opart_hbm.at[cid, pl.ds(sid * RPS, RPS)])


def _k5_body(x_ref, o0_ref, o1_ref, b_ref, out_ref):
    out_ref[...] = x_ref[...] + o0_ref[0] + o1_ref[0] + b_ref[...]


_R5 = 1000
_k5 = pl.pallas_call(
    _k5_body,
    grid=(N // _R5,),
    in_specs=[
        pl.BlockSpec((_R5, D), lambda i: (i, 0)),
        pl.BlockSpec((1, _R5, D), lambda i: (0, i, 0)),
        pl.BlockSpec((1, _R5, D), lambda i: (1, i, 0)),
        pl.BlockSpec((1, D), lambda i: (0, 0)),
    ],
    out_specs=pl.BlockSpec((_R5, D), lambda i: (i, 0)),
    out_shape=jax.ShapeDtypeStruct((N, D), jnp.float32),
)


def kernel(x, adj, W, att_l, att_r, bias, gamma, beta):
    Wt = W.T
    attl16 = jnp.pad(att_l, ((0, 0), (0, HP - H)))
    attr16 = jnp.pad(att_r, ((0, 0), (0, HP - H)))
    xp = jnp.pad(x, ((0, NP - N), (0, 0)))
    xl, al16, ar16, cal, car = _k1(xp, Wt, attl16, attr16,
                                   gamma.reshape(1, D), beta.reshape(1, D))
    pad = jnp.full((EPAD - E,), N, jnp.int32)
    dst = jnp.concatenate([adj[1], pad]).reshape(EPAD // EB, EB)
    src = jnp.concatenate([adj[0], pad]).reshape(EPAD // EB, EB)
    z16 = jnp.zeros((NP, HP), jnp.float32)
    ex, dpart = _k2(dst, src, al16, ar16, cal, car, z16)
    rcp = _k3(dpart)
    znd = jnp.zeros((NP, D), jnp.float32)
    opart = _k4(dst, src, xl, ex, rcp, znd)
    return _k5(x, opart, opart, bias.reshape(1, D))


# speedup vs baseline: 1.8300x; 1.8300x over previous
import functools

import jax
import jax.numpy as jnp
from jax import lax
from jax.experimental import pallas as pl
from jax.experimental.pallas import tpu as pltpu
from jax.experimental.pallas import tpu_sc as plsc

N = 10000
E = 320000
D = 128
H = 4
HP = 16

NC = 2
NS = 16
NW = NC * NS
EB = 128
NBW = 80
NBH = NBW // 4
EPAD = NW * NBW * EB
NP = 10240
RPS = NP // NS

_R1 = 1024
_G1 = NP // _R1

_mesh = plsc.VectorSubcoreMesh(core_axis_name="c", subcore_axis_name="s")
_sc_params = pltpu.CompilerParams(use_tc_tiling_on_sc=False,
                                  needs_layout_passes=False)


def _k1_body(x_ref, wt_ref, attl_ref, attr_ref, g_ref, b_ref,
             xl_ref, al_ref, ar_ref, cal_ref, car_ref):
    i = pl.program_id(0)
    x = x_ref[...]
    mu = jnp.mean(x, axis=-1, keepdims=True)
    xc = x - mu
    var = jnp.mean(xc * xc, axis=-1, keepdims=True)
    h = g_ref[...] * xc * lax.rsqrt(var + 1e-5) + b_ref[...]
    xl = jnp.dot(h, wt_ref[...], preferred_element_type=jnp.float32)
    xl_ref[...] = xl
    al = jnp.dot(xl, attl_ref[...], preferred_element_type=jnp.float32)
    ar = jnp.dot(xl, attr_ref[...], preferred_element_type=jnp.float32)
    al_ref[...] = al
    ar_ref[...] = ar
    bl = jnp.max(al, axis=0, keepdims=True)
    br = jnp.max(ar, axis=0, keepdims=True)

    @pl.when(i == 0)
    def _():
        cal_ref[...] = bl
        car_ref[...] = br

    @pl.when(i > 0)
    def _():
        cal_ref[...] = jnp.maximum(cal_ref[...], bl)
        car_ref[...] = jnp.maximum(car_ref[...], br)


_k1 = pl.pallas_call(
    _k1_body,
    grid=(_G1,),
    in_specs=[
        pl.BlockSpec((_R1, D), lambda i: (i, 0)),
        pl.BlockSpec((D, D), lambda i: (0, 0)),
        pl.BlockSpec((D, HP), lambda i: (0, 0)),
        pl.BlockSpec((D, HP), lambda i: (0, 0)),
        pl.BlockSpec((1, D), lambda i: (0, 0)),
        pl.BlockSpec((1, D), lambda i: (0, 0)),
    ],
    out_specs=[
        pl.BlockSpec((_R1, D), lambda i: (i, 0)),
        pl.BlockSpec((_R1, HP), lambda i: (i, 0)),
        pl.BlockSpec((_R1, HP), lambda i: (i, 0)),
        pl.BlockSpec((1, HP), lambda i: (0, 0)),
        pl.BlockSpec((1, HP), lambda i: (0, 0)),
    ],
    out_shape=[
        jax.ShapeDtypeStruct((NP, D), jnp.float32),
        jax.ShapeDtypeStruct((NP, HP), jnp.float32),
        jax.ShapeDtypeStruct((NP, HP), jnp.float32),
        jax.ShapeDtypeStruct((1, HP), jnp.float32),
        jax.ShapeDtypeStruct((1, HP), jnp.float32),
    ],
)


@functools.partial(
    pl.kernel,
    out_type=[
        jax.ShapeDtypeStruct((EPAD, HP), jnp.float32),
        jax.ShapeDtypeStruct((NC, NP, HP), jnp.float32),
    ],
    mesh=_mesh,
    scratch_types=[
        pltpu.VMEM((NBW, EB), jnp.int32),
        pltpu.VMEM((NBW, EB), jnp.int32),
        pltpu.VMEM((EB, HP), jnp.float32), pltpu.VMEM((EB, HP), jnp.float32),
        pltpu.VMEM((EB, HP), jnp.float32), pltpu.VMEM((EB, HP), jnp.float32),
        pltpu.VMEM((16,), jnp.float32),
        pltpu.VMEM((16,), jnp.float32),
        pltpu.VMEM_SHARED((NP, HP), jnp.float32),
        pltpu.SemaphoreType.DMA, pltpu.SemaphoreType.DMA,
    ],
    compiler_params=_sc_params,
)
def _k2(dst_hbm, src_hbm, al_hbm, ar_hbm, cal_hbm, car_hbm, z16_hbm,
        ex_hbm, dpart_hbm,
        dstv, srcv, gal0, gal1, gar0, gar1,
        calv, carv, dshared, sem0, sem1):
    cid = lax.axis_index("c")
    sid = lax.axis_index("s")
    wid = sid * NC + cid
    gal, gar = (gal0, gal1), (gar0, gar1)
    sem = (sem0, sem1)
    pltpu.sync_copy(z16_hbm.at[pl.ds(sid * RPS, RPS)],
                    dshared.at[pl.ds(sid * RPS, RPS)])
    pltpu.sync_copy(cal_hbm.at[0], calv)
    pltpu.sync_copy(car_hbm.at[0], carv)
    pltpu.sync_copy(dst_hbm.at[pl.ds(wid * NBW, NBW)], dstv)
    pltpu.sync_copy(src_hbm.at[pl.ds(wid * NBW, NBW)], srcv)
    plsc.subcore_barrier()
    c = jnp.maximum(calv[...] + carv[...], 0.0)
    e_base = wid * NBW * EB

    def issue(g, b):
        pltpu.async_copy(al_hbm.at[dstv.at[g]], gal[b], sem[b])
        pltpu.async_copy(ar_hbm.at[srcv.at[g]], gar[b], sem[b])

    def wait_in(g, b):
        pltpu.make_async_copy(al_hbm.at[dstv.at[g]], gal[b], sem[b]).wait()
        pltpu.make_async_copy(ar_hbm.at[srcv.at[g]], gar[b], sem[b]).wait()

    issue(0, 0)

    @pl.loop(0, NBW, step=2)
    def _(g):
        for b in (0, 1):
            gg = g + b
            wait_in(gg, b)

            @pl.when(gg + 1 < NBW)
            def _():
                issue(gg + 1, 1 - b)

            galb, garb = gal[b], gar[b]

            @plsc.parallel_loop(0, EB, unroll=4)
            def _(j):
                s = galb[j, :] + garb[j, :]
                s = jnp.maximum(s, 0.2 * s)
                galb[j, :] = jnp.exp(s - c)

            pltpu.sync_copy(galb, ex_hbm.at[pl.ds(e_base + gg * EB, EB)])
            pltpu.sync_copy(galb, dshared.at[dstv.at[gg]], add=True)

    plsc.subcore_barrier()
    pltpu.sync_copy(dshared.at[pl.ds(sid * RPS, RPS)],
                    dpart_hbm.at[cid, pl.ds(sid * RPS, RPS)])


def _k3_body(dp_ref, rcp_ref):
    d = dp_ref[0] + dp_ref[1]
    lanes = lax.broadcasted_iota(jnp.int32, (1, HP), 1)
    rcp_ref[...] = jnp.where(lanes < H, 1.0 / (d + 1e-16), 0.0)


_k3 = pl.pallas_call(
    _k3_body,
    out_shape=jax.ShapeDtypeStruct((NP, HP), jnp.float32),
)


@functools.partial(
    pl.kernel,
    out_type=jax.ShapeDtypeStruct((NC, NP, D), jnp.float32),
    mesh=_mesh,
    scratch_types=[
        pltpu.VMEM((EB,), jnp.int32), pltpu.VMEM((EB,), jnp.int32),
        pltpu.VMEM((EB,), jnp.int32), pltpu.VMEM((EB,), jnp.int32),
        pltpu.VMEM((EB, HP), jnp.float32), pltpu.VMEM((EB, HP), jnp.float32),
        pltpu.VMEM((EB, HP), jnp.float32), pltpu.VMEM((EB, HP), jnp.float32),
        pltpu.VMEM((EB, D), jnp.float32), pltpu.VMEM((EB, D), jnp.float32),
        pltpu.VMEM((EB,), jnp.float32),
        pltpu.VMEM_SHARED((NP, D), jnp.float32),
        pltpu.SemaphoreType.DMA, pltpu.SemaphoreType.DMA,
    ],
    compiler_params=_sc_params,
)
def _k4(dst_hbm, src_hbm, xl_hbm, ex_hbm, rcp_hbm, znd_hbm,
        opart_hbm,
        dsti0, dsti1, srci0, srci1, exb0, exb1, rcpb0, rcpb1,
        rows0, rows1, wbuf, oshared, sem0, sem1):
    cid = lax.axis_index("c")
    sid = lax.axis_index("s")
    wid = sid * NC + cid
    dsti, srci = (dsti0, dsti1), (srci0, srci1)
    exb, rcpb, rows = (exb0, exb1), (rcpb0, rcpb1), (rows0, rows1)
    sem = (sem0, sem1)
    pltpu.sync_copy(znd_hbm.at[pl.ds(sid * RPS, RPS)],
                    oshared.at[pl.ds(sid * RPS, RPS)])
    plsc.subcore_barrier()
    lane = lax.iota(jnp.int32, 16)
    colh = [jnp.full((16,), h, jnp.int32) for h in range(H)]
    blk0 = wid * NBW
    e_base = blk0 * EB

    def issue(g, b):
        pltpu.sync_copy(dst_hbm.at[blk0 + g], dsti[b])
        pltpu.sync_copy(src_hbm.at[blk0 + g], srci[b])
        pltpu.async_copy(ex_hbm.at[pl.ds(e_base + g * EB, EB)], exb[b], sem[b])
        pltpu.async_copy(rcp_hbm.at[dsti[b]], rcpb[b], sem[b])
        pltpu.async_copy(xl_hbm.at[srci[b]], rows[b], sem[b])

    def wait_in(g, b):
        pltpu.make_async_copy(ex_hbm.at[pl.ds(e_base + g * EB, EB)], exb[b],
                              sem[b]).wait()
        pltpu.make_async_copy(rcp_hbm.at[dsti[b]], rcpb[b], sem[b]).wait()
        pltpu.make_async_copy(xl_hbm.at[srci[b]], rows[b], sem[b]).wait()

    issue(0, 0)

    @pl.loop(0, NBW, step=2)
    def _(g):
        for b in (0, 1):
            gg = g + b
            wait_in(gg, b)

            @pl.when(gg + 1 < NBW)
            def _():
                issue(gg + 1, 1 - b)

            exbb, rcpbb, rowsb = exb[b], rcpb[b], rows[b]

            @pl.loop(0, EB, step=16)
            def _(j):
                rowi = lane + j
                acc = (plsc.load_gather(exbb, [rowi, colh[0]])
                       * plsc.load_gather(rcpbb, [rowi, colh[0]]))
                for h in range(1, H):
                    acc = acc + (plsc.load_gather(exbb, [rowi, colh[h]])
                                 * plsc.load_gather(rcpbb, [rowi, colh[h]]))
                wbuf[pl.ds(j, 16)] = 0.25 * acc

            @pl.loop(0, EB, unroll=2)
            def _(j):
                wv = plsc.load_gather(
                    wbuf, [jnp.full((16,), 0, jnp.int32) + j])
                for k in range(D // 16):
                    rowsb[j, pl.ds(k * 16, 16)] = (
                        rowsb[j, pl.ds(k * 16, 16)] * wv)

            pltpu.sync_copy(rowsb, oshared.at[dsti[b]], add=True)

    plsc.subcore_barrier()
    pltpu.sync_copy(oshared.at[pl.ds(sid * RPS, RPS)],
                    opart_hbm.at[cid, pl.ds(sid * RPS, RPS)])


def _k5_body(x_ref, o0_ref, o1_ref, b_ref, out_ref):
    out_ref[...] = x_ref[...] + o0_ref[0] + o1_ref[0] + b_ref[...]


_R5 = 1000
_k5 = pl.pallas_call(
    _k5_body,
    grid=(N // _R5,),
    in_specs=[
        pl.BlockSpec((_R5, D), lambda i: (i, 0)),
        pl.BlockSpec((1, _R5, D), lambda i: (0, i, 0)),
        pl.BlockSpec((1, _R5, D), lambda i: (1, i, 0)),
        pl.BlockSpec((1, D), lambda i: (0, 0)),
    ],
    out_specs=pl.BlockSpec((_R5, D), lambda i: (i, 0)),
    out_shape=jax.ShapeDtypeStruct((N, D), jnp.float32),
)


def kernel(x, adj, W, att_l, att_r, bias, gamma, beta):
    Wt = W.T
    attl16 = jnp.pad(att_l, ((0, 0), (0, HP - H)))
    attr16 = jnp.pad(att_r, ((0, 0), (0, HP - H)))
    xp = jnp.pad(x, ((0, NP - N), (0, 0)))
    xl, al16, ar16, cal, car = _k1(xp, Wt, attl16, attr16,
                                   gamma.reshape(1, D), beta.reshape(1, D))
    pad = N + (jnp.arange(EPAD - E, dtype=jnp.int32) % (NP - N))
    dst = jnp.concatenate([adj[1], pad]).reshape(EPAD // EB, EB)
    src = jnp.concatenate([adj[0], pad]).reshape(EPAD // EB, EB)
    z16 = jnp.zeros((NP, HP), jnp.float32)
    ex, dpart = _k2(dst, src, al16, ar16, cal, car, z16)
    rcp = _k3(dpart)
    znd = jnp.zeros((NP, D), jnp.float32)
    opart = _k4(dst, src, xl, ex, rcp, znd)
    return _k5(x, opart, opart, bias.reshape(1, D))


# speedup vs baseline: 2.0139x; 1.1005x over previous
import functools

import jax
import jax.numpy as jnp
from jax import lax
from jax.experimental import pallas as pl
from jax.experimental.pallas import tpu as pltpu
from jax.experimental.pallas import tpu_sc as plsc

N = 10000
E = 320000
D = 128
H = 4
HP = 16

NC = 2
NS = 16
NW = NC * NS
EB = 128
NBW = 80
NBH = NBW // 4
EPAD = NW * NBW * EB
NP = 10240
RPS = NP // NS

_R1 = 1024
_G1 = NP // _R1

_mesh = plsc.VectorSubcoreMesh(core_axis_name="c", subcore_axis_name="s")
_sc_params = pltpu.CompilerParams(use_tc_tiling_on_sc=False,
                                  needs_layout_passes=False)


def _k1_body(x_ref, wt_ref, attl_ref, attr_ref, g_ref, b_ref,
             xl_ref, al_ref, ar_ref, cal_ref, car_ref):
    i = pl.program_id(0)
    x = x_ref[...]
    mu = jnp.mean(x, axis=-1, keepdims=True)
    xc = x - mu
    var = jnp.mean(xc * xc, axis=-1, keepdims=True)
    h = g_ref[...] * xc * lax.rsqrt(var + 1e-5) + b_ref[...]
    xl = jnp.dot(h, wt_ref[...], preferred_element_type=jnp.float32)
    xl_ref[...] = xl
    al = jnp.dot(xl, attl_ref[...], preferred_element_type=jnp.float32)
    ar = jnp.dot(xl, attr_ref[...], preferred_element_type=jnp.float32)
    al_ref[...] = al
    ar_ref[...] = ar
    bl = jnp.max(al, axis=0, keepdims=True)
    br = jnp.max(ar, axis=0, keepdims=True)

    @pl.when(i == 0)
    def _():
        cal_ref[...] = bl
        car_ref[...] = br

    @pl.when(i > 0)
    def _():
        cal_ref[...] = jnp.maximum(cal_ref[...], bl)
        car_ref[...] = jnp.maximum(car_ref[...], br)


_k1 = pl.pallas_call(
    _k1_body,
    grid=(_G1,),
    in_specs=[
        pl.BlockSpec((_R1, D), lambda i: (i, 0)),
        pl.BlockSpec((D, D), lambda i: (0, 0)),
        pl.BlockSpec((D, HP), lambda i: (0, 0)),
        pl.BlockSpec((D, HP), lambda i: (0, 0)),
        pl.BlockSpec((1, D), lambda i: (0, 0)),
        pl.BlockSpec((1, D), lambda i: (0, 0)),
    ],
    out_specs=[
        pl.BlockSpec((_R1, D), lambda i: (i, 0)),
        pl.BlockSpec((_R1, HP), lambda i: (i, 0)),
        pl.BlockSpec((_R1, HP), lambda i: (i, 0)),
        pl.BlockSpec((1, HP), lambda i: (0, 0)),
        pl.BlockSpec((1, HP), lambda i: (0, 0)),
    ],
    out_shape=[
        jax.ShapeDtypeStruct((NP, D), jnp.float32),
        jax.ShapeDtypeStruct((NP, HP), jnp.float32),
        jax.ShapeDtypeStruct((NP, HP), jnp.float32),
        jax.ShapeDtypeStruct((1, HP), jnp.float32),
        jax.ShapeDtypeStruct((1, HP), jnp.float32),
    ],
)


@functools.partial(
    pl.kernel,
    out_type=[
        jax.ShapeDtypeStruct((EPAD, HP), jnp.float32),
        jax.ShapeDtypeStruct((NC, NP, HP), jnp.float32),
    ],
    mesh=_mesh,
    scratch_types=[
        pltpu.VMEM((NBW, EB), jnp.int32),
        pltpu.VMEM((NBW, EB), jnp.int32),
        pltpu.VMEM((EB, HP), jnp.float32), pltpu.VMEM((EB, HP), jnp.float32),
        pltpu.VMEM((EB, HP), jnp.float32), pltpu.VMEM((EB, HP), jnp.float32),
        pltpu.VMEM((16,), jnp.float32),
        pltpu.VMEM((16,), jnp.float32),
        pltpu.VMEM_SHARED((NP, HP), jnp.float32),
        pltpu.SemaphoreType.DMA, pltpu.SemaphoreType.DMA,
    ],
    compiler_params=_sc_params,
)
def _k2(dst_hbm, src_hbm, al_hbm, ar_hbm, cal_hbm, car_hbm, z16_hbm,
        ex_hbm, dpart_hbm,
        dstv, srcv, gal0, gal1, gar0, gar1,
        calv, carv, dshared, sem0, sem1):
    cid = lax.axis_index("c")
    sid = lax.axis_index("s")
    wid = sid * NC + cid
    gal, gar = (gal0, gal1), (gar0, gar1)
    sem = (sem0, sem1)
    pltpu.sync_copy(z16_hbm.at[pl.ds(sid * RPS, RPS)],
                    dshared.at[pl.ds(sid * RPS, RPS)])
    pltpu.sync_copy(cal_hbm.at[0], calv)
    pltpu.sync_copy(car_hbm.at[0], carv)
    pltpu.sync_copy(dst_hbm.at[pl.ds(wid * NBW, NBW)], dstv)
    pltpu.sync_copy(src_hbm.at[pl.ds(wid * NBW, NBW)], srcv)
    plsc.subcore_barrier()
    c = jnp.maximum(calv[...] + carv[...], 0.0)
    e_base = wid * NBW * EB

    def issue(g, b):
        pltpu.async_copy(al_hbm.at[dstv.at[g]], gal[b], sem[b])
        pltpu.async_copy(ar_hbm.at[srcv.at[g]], gar[b], sem[b])

    def wait_in(g, b):
        pltpu.make_async_copy(al_hbm.at[dstv.at[g]], gal[b], sem[b]).wait()
        pltpu.make_async_copy(ar_hbm.at[srcv.at[g]], gar[b], sem[b]).wait()

    issue(0, 0)

    @pl.loop(0, NBW, step=2)
    def _(g):
        for b in (0, 1):
            gg = g + b
            wait_in(gg, b)

            @pl.when(gg + 1 < NBW)
            def _():
                issue(gg + 1, 1 - b)

            galb, garb = gal[b], gar[b]

            @plsc.parallel_loop(0, EB, unroll=4)
            def _(j):
                s = galb[j, :] + garb[j, :]
                s = jnp.maximum(s, 0.2 * s)
                galb[j, :] = jnp.exp(s - c)

            pltpu.sync_copy(galb, ex_hbm.at[pl.ds(e_base + gg * EB, EB)])
            pltpu.sync_copy(galb, dshared.at[dstv.at[gg]], add=True)

    plsc.subcore_barrier()
    pltpu.sync_copy(dshared.at[pl.ds(sid * RPS, RPS)],
                    dpart_hbm.at[cid, pl.ds(sid * RPS, RPS)])


def _k3_body(dp_ref, rcp_ref):
    d = dp_ref[0] + dp_ref[1]
    lanes = lax.broadcasted_iota(jnp.int32, (1, HP), 1)
    rcp_ref[...] = jnp.where(lanes < H, 1.0 / (d + 1e-16), 0.0)


_k3 = pl.pallas_call(
    _k3_body,
    out_shape=jax.ShapeDtypeStruct((NP, HP), jnp.float32),
)


@functools.partial(
    pl.kernel,
    out_type=jax.ShapeDtypeStruct((NC, NP, D), jnp.float32),
    mesh=_mesh,
    scratch_types=[
        pltpu.VMEM((EB,), jnp.int32), pltpu.VMEM((EB,), jnp.int32),
        pltpu.VMEM((EB,), jnp.int32), pltpu.VMEM((EB,), jnp.int32),
        pltpu.VMEM((EB, HP), jnp.float32), pltpu.VMEM((EB, HP), jnp.float32),
        pltpu.VMEM((EB, HP), jnp.float32), pltpu.VMEM((EB, HP), jnp.float32),
        pltpu.VMEM((EB, D), jnp.float32), pltpu.VMEM((EB, D), jnp.float32),
        pltpu.VMEM((EB,), jnp.float32),
        pltpu.VMEM_SHARED((NP, D), jnp.float32),
        pltpu.SemaphoreType.DMA, pltpu.SemaphoreType.DMA,
    ],
    compiler_params=_sc_params,
)
def _k4(dst_hbm, src_hbm, xl_hbm, ex_hbm, rcp_hbm, znd_hbm,
        opart_hbm,
        dsti0, dsti1, srci0, srci1, exb0, exb1, rcpb0, rcpb1,
        rows0, rows1, wbuf, oshared, sem0, sem1):
    cid = lax.axis_index("c")
    sid = lax.axis_index("s")
    wid = sid * NC + cid
    dsti, srci = (dsti0, dsti1), (srci0, srci1)
    exb, rcpb, rows = (exb0, exb1), (rcpb0, rcpb1), (rows0, rows1)
    sem = (sem0, sem1)
    pltpu.sync_copy(znd_hbm.at[pl.ds(sid * RPS, RPS)],
                    oshared.at[pl.ds(sid * RPS, RPS)])
    plsc.subcore_barrier()
    lane = lax.iota(jnp.int32, 16)
    colh = [jnp.full((16,), h, jnp.int32) for h in range(H)]
    blk0 = wid * NBW
    e_base = blk0 * EB

    def issue(g, b):
        pltpu.sync_copy(dst_hbm.at[blk0 + g], dsti[b])
        pltpu.sync_copy(src_hbm.at[blk0 + g], srci[b])
        pltpu.async_copy(ex_hbm.at[pl.ds(e_base + g * EB, EB)], exb[b], sem[b])
        pltpu.async_copy(rcp_hbm.at[dsti[b]], rcpb[b], sem[b])
        pltpu.async_copy(xl_hbm.at[srci[b]], rows[b], sem[b])

    def wait_in(g, b):
        pltpu.make_async_copy(ex_hbm.at[pl.ds(e_base + g * EB, EB)], exb[b],
                              sem[b]).wait()
        pltpu.make_async_copy(rcp_hbm.at[dsti[b]], rcpb[b], sem[b]).wait()
        pltpu.make_async_copy(xl_hbm.at[srci[b]], rows[b], sem[b]).wait()

    issue(0, 0)

    @pl.loop(0, NBW, step=2)
    def _(g):
        for b in (0, 1):
            gg = g + b
            wait_in(gg, b)

            @pl.when(gg + 1 < NBW)
            def _():
                issue(gg + 1, 1 - b)

            exbb, rcpbb, rowsb = exb[b], rcpb[b], rows[b]

            @plsc.parallel_loop(0, EB, step=16, unroll=2)
            def _(j):
                rowi = lane + j
                acc = (plsc.load_gather(exbb, [rowi, colh[0]])
                       * plsc.load_gather(rcpbb, [rowi, colh[0]]))
                for h in range(1, H):
                    acc = acc + (plsc.load_gather(exbb, [rowi, colh[h]])
                                 * plsc.load_gather(rcpbb, [rowi, colh[h]]))
                wbuf[pl.ds(j, 16)] = 0.25 * acc

            @plsc.parallel_loop(0, EB, unroll=2)
            def _(j):
                wv = plsc.load_gather(
                    wbuf, [jnp.full((16,), 0, jnp.int32) + j])
                for k in range(D // 16):
                    rowsb[j, pl.ds(k * 16, 16)] = (
                        rowsb[j, pl.ds(k * 16, 16)] * wv)

            pltpu.sync_copy(rowsb, oshared.at[dsti[b]], add=True)

    plsc.subcore_barrier()
    pltpu.sync_copy(oshared.at[pl.ds(sid * RPS, RPS)],
                    opart_hbm.at[cid, pl.ds(sid * RPS, RPS)])


def _k5_body(x_ref, o0_ref, o1_ref, b_ref, out_ref):
    out_ref[...] = x_ref[...] + o0_ref[0] + o1_ref[0] + b_ref[...]


_R5 = 1000
_k5 = pl.pallas_call(
    _k5_body,
    grid=(N // _R5,),
    in_specs=[
        pl.BlockSpec((_R5, D), lambda i: (i, 0)),
        pl.BlockSpec((1, _R5, D), lambda i: (0, i, 0)),
        pl.BlockSpec((1, _R5, D), lambda i: (1, i, 0)),
        pl.BlockSpec((1, D), lambda i: (0, 0)),
    ],
    out_specs=pl.BlockSpec((_R5, D), lambda i: (i, 0)),
    out_shape=jax.ShapeDtypeStruct((N, D), jnp.float32),
)


def kernel(x, adj, W, att_l, att_r, bias, gamma, beta):
    Wt = W.T
    attl16 = jnp.pad(att_l, ((0, 0), (0, HP - H)))
    attr16 = jnp.pad(att_r, ((0, 0), (0, HP - H)))
    xp = jnp.pad(x, ((0, NP - N), (0, 0)))
    xl, al16, ar16, cal, car = _k1(xp, Wt, attl16, attr16,
                                   gamma.reshape(1, D), beta.reshape(1, D))
    pad = N + (jnp.arange(EPAD - E, dtype=jnp.int32) % (NP - N))
    dst = jnp.concatenate([adj[1], pad]).reshape(EPAD // EB, EB)
    src = jnp.concatenate([adj[0], pad]).reshape(EPAD // EB, EB)
    z16 = jnp.zeros((NP, HP), jnp.float32)
    ex, dpart = _k2(dst, src, al16, ar16, cal, car, z16)
    rcp = _k3(dpart)
    znd = jnp.zeros((NP, D), jnp.float32)
    opart = _k4(dst, src, xl, ex, rcp, znd)
    return _k5(x, opart, opart, bias.reshape(1, D))


# speedup vs baseline: 2.0145x; 1.0003x over previous
import functools

import jax
import jax.numpy as jnp
from jax import lax
from jax.experimental import pallas as pl
from jax.experimental.pallas import tpu as pltpu
from jax.experimental.pallas import tpu_sc as plsc

N = 10000
E = 320000
D = 128
H = 4
HP = 16

NC = 2
NS = 16
NW = NC * NS
EB = 128
NBW = 80
NBH = NBW // 4
EPAD = NW * NBW * EB
NP = 10240
RPS = NP // NS

_R1 = 1024
_G1 = NP // _R1

_mesh = plsc.VectorSubcoreMesh(core_axis_name="c", subcore_axis_name="s")
_sc_params = pltpu.CompilerParams(use_tc_tiling_on_sc=False,
                                  needs_layout_passes=False)


def _k1_body(x_ref, wt_ref, attl_ref, attr_ref, g_ref, b_ref,
             xl_ref, al_ref, ar_ref, cal_ref, car_ref):
    i = pl.program_id(0)
    x = x_ref[...]
    mu = jnp.mean(x, axis=-1, keepdims=True)
    xc = x - mu
    var = jnp.mean(xc * xc, axis=-1, keepdims=True)
    h = g_ref[...] * xc * lax.rsqrt(var + 1e-5) + b_ref[...]
    xl = jnp.dot(h, wt_ref[...], preferred_element_type=jnp.float32)
    xl_ref[...] = xl
    al = jnp.dot(xl, attl_ref[...], preferred_element_type=jnp.float32)
    ar = jnp.dot(xl, attr_ref[...], preferred_element_type=jnp.float32)
    al_ref[...] = al
    ar_ref[...] = ar
    bl = jnp.max(al, axis=0, keepdims=True)
    br = jnp.max(ar, axis=0, keepdims=True)

    @pl.when(i == 0)
    def _():
        cal_ref[...] = bl
        car_ref[...] = br

    @pl.when(i > 0)
    def _():
        cal_ref[...] = jnp.maximum(cal_ref[...], bl)
        car_ref[...] = jnp.maximum(car_ref[...], br)


_k1 = pl.pallas_call(
    _k1_body,
    grid=(_G1,),
    in_specs=[
        pl.BlockSpec((_R1, D), lambda i: (i, 0)),
        pl.BlockSpec((D, D), lambda i: (0, 0)),
        pl.BlockSpec((D, HP), lambda i: (0, 0)),
        pl.BlockSpec((D, HP), lambda i: (0, 0)),
        pl.BlockSpec((1, D), lambda i: (0, 0)),
        pl.BlockSpec((1, D), lambda i: (0, 0)),
    ],
    out_specs=[
        pl.BlockSpec((_R1, D), lambda i: (i, 0)),
        pl.BlockSpec((_R1, HP), lambda i: (i, 0)),
        pl.BlockSpec((_R1, HP), lambda i: (i, 0)),
        pl.BlockSpec((1, HP), lambda i: (0, 0)),
        pl.BlockSpec((1, HP), lambda i: (0, 0)),
    ],
    out_shape=[
        jax.ShapeDtypeStruct((NP, D), jnp.float32),
        jax.ShapeDtypeStruct((NP, HP), jnp.float32),
        jax.ShapeDtypeStruct((NP, HP), jnp.float32),
        jax.ShapeDtypeStruct((1, HP), jnp.float32),
        jax.ShapeDtypeStruct((1, HP), jnp.float32),
    ],
)


@functools.partial(
    pl.kernel,
    out_type=[
        jax.ShapeDtypeStruct((EPAD, HP), jnp.float32),
        jax.ShapeDtypeStruct((NC, NP, HP), jnp.float32),
    ],
    mesh=_mesh,
    scratch_types=[
        pltpu.VMEM((NBW, EB), jnp.int32),
        pltpu.VMEM((NBW, EB), jnp.int32),
        pltpu.VMEM((EB, HP), jnp.float32), pltpu.VMEM((EB, HP), jnp.float32),
        pltpu.VMEM((EB, HP), jnp.float32), pltpu.VMEM((EB, HP), jnp.float32),
        pltpu.VMEM((16,), jnp.float32),
        pltpu.VMEM((16,), jnp.float32),
        pltpu.VMEM_SHARED((NP, HP), jnp.float32),
        pltpu.SemaphoreType.DMA, pltpu.SemaphoreType.DMA,
    ],
    compiler_params=_sc_params,
)
def _k2(dst_hbm, src_hbm, al_hbm, ar_hbm, cal_hbm, car_hbm, z16_hbm,
        ex_hbm, dpart_hbm,
        dstv, srcv, gal0, gal1, gar0, gar1,
        calv, carv, dshared, sem0, sem1):
    cid = lax.axis_index("c")
    sid = lax.axis_index("s")
    wid = sid * NC + cid
    gal, gar = (gal0, gal1), (gar0, gar1)
    sem = (sem0, sem1)
    pltpu.sync_copy(z16_hbm.at[pl.ds(sid * RPS, RPS)],
                    dshared.at[pl.ds(sid * RPS, RPS)])
    pltpu.sync_copy(cal_hbm.at[0], calv)
    pltpu.sync_copy(car_hbm.at[0], carv)
    pltpu.sync_copy(dst_hbm.at[pl.ds(wid * NBW, NBW)], dstv)
    pltpu.sync_copy(src_hbm.at[pl.ds(wid * NBW, NBW)], srcv)
    plsc.subcore_barrier()
    c = jnp.maximum(calv[...] + carv[...], 0.0)
    e_base = wid * NBW * EB

    def issue(g, b):
        pltpu.async_copy(al_hbm.at[dstv.at[g]], gal[b], sem[b])
        pltpu.async_copy(ar_hbm.at[srcv.at[g]], gar[b], sem[b])

    def wait_in(g, b):
        pltpu.make_async_copy(al_hbm.at[dstv.at[g]], gal[b], sem[b]).wait()
        pltpu.make_async_copy(ar_hbm.at[srcv.at[g]], gar[b], sem[b]).wait()

    issue(0, 0)

    @pl.loop(0, NBW, step=2)
    def _(g):
        for b in (0, 1):
            gg = g + b
            wait_in(gg, b)

            @pl.when(gg + 1 < NBW)
            def _():
                issue(gg + 1, 1 - b)

            galb, garb = gal[b], gar[b]

            @plsc.parallel_loop(0, EB, unroll=4)
            def _(j):
                s = galb[j, :] + garb[j, :]
                s = jnp.maximum(s, 0.2 * s)
                galb[j, :] = jnp.exp(s - c)

            pltpu.sync_copy(galb, ex_hbm.at[pl.ds(e_base + gg * EB, EB)])
            pltpu.sync_copy(galb, dshared.at[dstv.at[gg]], add=True)

    plsc.subcore_barrier()
    pltpu.sync_copy(dshared.at[pl.ds(sid * RPS, RPS)],
                    dpart_hbm.at[cid, pl.ds(sid * RPS, RPS)])


def _k3_body(dp_ref, rcp_ref):
    d = dp_ref[0] + dp_ref[1]
    lanes = lax.broadcasted_iota(jnp.int32, (1, HP), 1)
    rcp_ref[...] = jnp.where(lanes < H, 1.0 / (d + 1e-16), 0.0)


_k3 = pl.pallas_call(
    _k3_body,
    out_shape=jax.ShapeDtypeStruct((NP, HP), jnp.float32),
)


@functools.partial(
    pl.kernel,
    out_type=jax.ShapeDtypeStruct((NC, NP, D), jnp.float32),
    mesh=_mesh,
    scratch_types=[
        pltpu.VMEM((EB,), jnp.int32), pltpu.VMEM((EB,), jnp.int32),
        pltpu.VMEM((EB,), jnp.int32), pltpu.VMEM((EB,), jnp.int32),
        pltpu.VMEM((EB, HP), jnp.float32), pltpu.VMEM((EB, HP), jnp.float32),
        pltpu.VMEM((EB, HP), jnp.float32), pltpu.VMEM((EB, HP), jnp.float32),
        pltpu.VMEM((EB, D), jnp.float32), pltpu.VMEM((EB, D), jnp.float32),
        pltpu.VMEM((EB,), jnp.float32),
        pltpu.VMEM_SHARED((NP, D), jnp.float32),
        pltpu.SemaphoreType.DMA, pltpu.SemaphoreType.DMA,
    ],
    compiler_params=_sc_params,
)
def _k4(dst_hbm, src_hbm, xl_hbm, ex_hbm, rcp_hbm, znd_hbm,
        opart_hbm,
        dsti0, dsti1, srci0, srci1, exb0, exb1, rcpb0, rcpb1,
        rows0, rows1, wbuf, oshared, sem0, sem1):
    cid = lax.axis_index("c")
    sid = lax.axis_index("s")
    wid = sid * NC + cid
    dsti, srci = (dsti0, dsti1), (srci0, srci1)
    exb, rcpb, rows = (exb0, exb1), (rcpb0, rcpb1), (rows0, rows1)
    sem = (sem0, sem1)
    pltpu.sync_copy(znd_hbm.at[pl.ds(sid * RPS, RPS)],
                    oshared.at[pl.ds(sid * RPS, RPS)])
    plsc.subcore_barrier()
    lane = lax.iota(jnp.int32, 16)
    colh = [jnp.full((16,), h, jnp.int32) for h in range(H)]
    blk0 = wid * NBW
    e_base = blk0 * EB

    def issue(g, b):
        pltpu.sync_copy(dst_hbm.at[blk0 + g], dsti[b])
        pltpu.sync_copy(src_hbm.at[blk0 + g], srci[b])
        pltpu.async_copy(ex_hbm.at[pl.ds(e_base + g * EB, EB)], exb[b], sem[b])
        pltpu.async_copy(rcp_hbm.at[dsti[b]], rcpb[b], sem[b])
        pltpu.async_copy(xl_hbm.at[srci[b]], rows[b], sem[b])

    def wait_in(g, b):
        pltpu.make_async_copy(ex_hbm.at[pl.ds(e_base + g * EB, EB)], exb[b],
                              sem[b]).wait()
        pltpu.make_async_copy(rcp_hbm.at[dsti[b]], rcpb[b], sem[b]).wait()
        pltpu.make_async_copy(xl_hbm.at[srci[b]], rows[b], sem[b]).wait()

    issue(0, 0)

    @pl.loop(0, NBW, step=2)
    def _(g):
        for b in (0, 1):
            gg = g + b
            wait_in(gg, b)

            @pl.when(gg + 1 < NBW)
            def _():
                issue(gg + 1, 1 - b)

            exbb, rcpbb, rowsb = exb[b], rcpb[b], rows[b]

            @plsc.parallel_loop(0, EB, step=16, unroll=2)
            def _(j):
                rowi = lane + j
                acc = (plsc.load_gather(exbb, [rowi, colh[0]])
                       * plsc.load_gather(rcpbb, [rowi, colh[0]]))
                for h in range(1, H):
                    acc = acc + (plsc.load_gather(exbb, [rowi, colh[h]])
                                 * plsc.load_gather(rcpbb, [rowi, colh[h]]))
                wbuf[pl.ds(j, 16)] = 0.25 * acc

            @plsc.parallel_loop(0, EB, unroll=4)
            def _(j):
                wv = plsc.load_gather(
                    wbuf, [jnp.full((16,), 0, jnp.int32) + j])
                for k in range(D // 16):
                    rowsb[j, pl.ds(k * 16, 16)] = (
                        rowsb[j, pl.ds(k * 16, 16)] * wv)

            pltpu.sync_copy(rowsb, oshared.at[dsti[b]], add=True)

    plsc.subcore_barrier()
    pltpu.sync_copy(oshared.at[pl.ds(sid * RPS, RPS)],
                    opart_hbm.at[cid, pl.ds(sid * RPS, RPS)])


def _k5_body(x_ref, o0_ref, o1_ref, b_ref, out_ref):
    out_ref[...] = x_ref[...] + o0_ref[0] + o1_ref[0] + b_ref[...]


_R5 = 1000
_k5 = pl.pallas_call(
    _k5_body,
    grid=(N // _R5,),
    in_specs=[
        pl.BlockSpec((_R5, D), lambda i: (i, 0)),
        pl.BlockSpec((1, _R5, D), lambda i: (0, i, 0)),
        pl.BlockSpec((1, _R5, D), lambda i: (1, i, 0)),
        pl.BlockSpec((1, D), lambda i: (0, 0)),
    ],
    out_specs=pl.BlockSpec((_R5, D), lambda i: (i, 0)),
    out_shape=jax.ShapeDtypeStruct((N, D), jnp.float32),
)


def kernel(x, adj, W, att_l, att_r, bias, gamma, beta):
    Wt = W.T
    attl16 = jnp.pad(att_l, ((0, 0), (0, HP - H)))
    attr16 = jnp.pad(att_r, ((0, 0), (0, HP - H)))
    xp = jnp.pad(x, ((0, NP - N), (0, 0)))
    xl, al16, ar16, cal, car = _k1(xp, Wt, attl16, attr16,
                                   gamma.reshape(1, D), beta.reshape(1, D))
    pad = N + (jnp.arange(EPAD - E, dtype=jnp.int32) % (NP - N))
    dst = jnp.concatenate([adj[1], pad]).reshape(EPAD // EB, EB)
    src = jnp.concatenate([adj[0], pad]).reshape(EPAD // EB, EB)
    z16 = jnp.zeros((NP, HP), jnp.float32)
    ex, dpart = _k2(dst, src, al16, ar16, cal, car, z16)
    rcp = _k3(dpart)
    znd = jnp.zeros((NP, D), jnp.float32)
    opart = _k4(dst, src, xl, ex, rcp, znd)
    return _k5(x, opart, opart, bias.reshape(1, D))
